# baseline (speedup 1.0000x reference)
import functools

import jax
import jax.numpy as jnp
from jax import lax
from jax.experimental import pallas as pl
from jax.experimental.pallas import tpu as pltpu

_MARGIN = 1.0


def _triplet_kernel(e_ref, tcol_ref, trow_ref, out_ref, d_ref, pm_ref, nm_ref):
    n = e_ref.shape[0]
    e = e_ref[:]
    g = lax.dot_general(e, e, (((1,), (1,)), ((), ())),
                        preferred_element_type=jnp.float32)
    r = jnp.sum(e * e, axis=1, keepdims=True)
    d = r + jnp.transpose(r) - 2.0 * g
    d_ref[:] = d

    tc = tcol_ref[:]
    tr = trow_ref[:]
    same = tc == tr
    rowid = lax.broadcasted_iota(jnp.int32, (n, n), 0)
    colid = lax.broadcasted_iota(jnp.int32, (n, n), 1)
    pm = (same & (rowid < colid)).astype(jnp.float32)
    nm = (~same).astype(jnp.float32)
    pm_ref[:] = pm
    nm_ref[:] = nm

    pm_rows = jnp.sum(pm, axis=1, keepdims=True)
    nm_rows = jnp.sum(nm, axis=1, keepdims=True)
    count = jnp.sum(pm_rows * nm_rows)

    nblk = n // 8

    def body(b, acc):
        rows = d_ref[pl.ds(b * 8, 8), :]
        pmr = pm_ref[pl.ds(b * 8, 8), :]
        nmr = nm_ref[pl.ds(b * 8, 8), :]
        cols1 = jnp.transpose(rows) + _MARGIN
        pmc = jnp.transpose(pmr)
        s = acc
        for j in range(8):
            m = jnp.maximum(cols1[:, j:j + 1] - rows[j:j + 1, :], 0.0)
            w = jnp.sum(m * nmr[j:j + 1, :], axis=1, keepdims=True)
            s = s + jnp.sum(w * pmc[:, j:j + 1])
        return s

    total = lax.fori_loop(0, nblk, body, jnp.float32(0.0))
    out_ref[:] = jnp.reshape(total / count, (1, 1))


@jax.jit
def kernel(embeddings, target):
    n, _ = embeddings.shape
    tcol = target.reshape(n, 1)
    trow = target.reshape(1, n)
    out = pl.pallas_call(
        _triplet_kernel,
        out_shape=jax.ShapeDtypeStruct((1, 1), jnp.float32),
        scratch_shapes=[
            pltpu.VMEM((n, n), jnp.float32),
            pltpu.VMEM((n, n), jnp.float32),
            pltpu.VMEM((n, n), jnp.float32),
        ],
    )(embeddings, tcol, trow)
    return out[0, 0]


# speedup vs baseline: 4.2203x; 4.2203x over previous
import jax
import jax.numpy as jnp
from jax import lax
from jax.experimental import pallas as pl
from jax.experimental.pallas import tpu as pltpu

_MARGIN = 1.0
_NEG_BIG = -3e38


def _triplet_kernel(e_ref, tcol_ref, trow_ref, out_ref, d_ref, pm_ref, nm_ref):
    n = e_ref.shape[0]
    e = e_ref[:]
    g = lax.dot_general(e, e, (((1,), (1,)), ((), ())),
                        preferred_element_type=jnp.float32)
    r = jnp.sum(e * e, axis=1, keepdims=True)
    d = r + jnp.transpose(r) - 2.0 * g
    d_ref[:] = d

    tc = tcol_ref[:]
    tr = trow_ref[:]
    same = tc == tr
    rowid = lax.broadcasted_iota(jnp.int32, (n, n), 0)
    colid = lax.broadcasted_iota(jnp.int32, (n, n), 1)
    pm = (same & (rowid < colid)).astype(jnp.float32)
    nm = (~same).astype(jnp.float32)
    pm_ref[:] = pm
    nm_ref[:] = nm

    pm_rows = jnp.sum(pm, axis=1, keepdims=True)
    nm_rows = jnp.sum(nm, axis=1, keepdims=True)
    count = jnp.sum(pm_rows * nm_rows)

    nsuper = 4
    sb = n // nsuper
    nblk = sb // 8

    acc8 = jnp.zeros((8, n), jnp.float32)
    for supb in range(nsuper):
        lo = supb * sb

        def body(k, acc, lo=lo):
            base = lo + k * 8
            rows = d_ref[pl.ds(base, 8), :]
            pmr = pm_ref[pl.ds(base, 8), :]
            nmr = nm_ref[pl.ds(base, 8), :]
            cols = jnp.transpose(rows)
            pmc = jnp.transpose(pmr)
            colsm = jnp.where(pmc != 0.0, cols + _MARGIN, _NEG_BIG)
            colsm = colsm[lo:, :]
            blk = jnp.zeros((n - lo, n), jnp.float32)
            for j in range(8):
                u = colsm[:, j:j + 1] - rows[j:j + 1, :]
                blk = blk + jnp.maximum(u * nmr[j:j + 1, :], 0.0)
            s = blk[0:8, :]
            for t in range(1, (n - lo) // 8):
                s = s + blk[t * 8:t * 8 + 8, :]
            return acc + s

        acc8 = lax.fori_loop(0, nblk, body, acc8)

    total = jnp.sum(acc8)
    out_ref[:] = jnp.reshape(total / count, (1, 1))


@jax.jit
def kernel(embeddings, target):
    n, _ = embeddings.shape
    tcol = target.reshape(n, 1)
    trow = target.reshape(1, n)
    out = pl.pallas_call(
        _triplet_kernel,
        out_shape=jax.ShapeDtypeStruct((1, 1), jnp.float32),
        scratch_shapes=[
            pltpu.VMEM((n, n), jnp.float32),
            pltpu.VMEM((n, n), jnp.float32),
            pltpu.VMEM((n, n), jnp.float32),
        ],
    )(embeddings, tcol, trow)
    return out[0, 0]
